# Initial kernel scaffold; baseline (speedup 1.0000x reference)
import jax
import jax.numpy as jnp
from jax.experimental import pallas as pl


def kernel(x, W1, g1, b1, W2, g2, b2, W3, g3, b3, fc1_w, fc1_b, fc2_w, fc2_b):
    raise NotImplementedError("write your pallas kernel here")


# speedup vs baseline: 2.1151x; 2.1151x over previous
import jax
import jax.numpy as jnp
from jax.experimental import pallas as pl
from jax.experimental.pallas import tpu as pltpu

_B, _N, _K = 16, 256, 20
_EPS = 1e-5
_NEG = -3.0e38


def _pairwise(xb):
    inner = jnp.dot(xb, xb.T, preferred_element_type=jnp.float32)
    xx = jnp.sum(xb * xb, axis=1)
    return 2.0 * inner - xx[:, None] - xx[None, :]


def _vndgcnn_kernel(x_ref, W1_ref, g1_ref, b1_ref, W2_ref, g2_ref, b2_ref,
                    W3_ref, g3_ref, b3_ref, fc1w_ref, fc1b_ref, fc2w_ref,
                    fc2b_ref, out_ref, cnt_ref, zbuf_ref):
    N, K = _N, _K
    iota = jax.lax.broadcasted_iota(jnp.int32, (N, N), 1)

    def phase1(b, carry):
        xb = x_ref[pl.ds(b, 1)].reshape(N, 8)
        P = _pairwise(xb)

        def step(_, mask):
            cur = jnp.where(mask > 0.5, _NEG, P)
            vmax = jnp.max(cur, axis=1, keepdims=True)
            ii = jnp.min(jnp.where(cur >= vmax, iota, N), axis=1,
                         keepdims=True)
            return mask + (iota == ii).astype(jnp.float32)

        mask = jax.lax.fori_loop(0, K, step, jnp.zeros((N, N), jnp.float32))
        cnt_ref[pl.ds(b, 1), :] = jnp.sum(mask, axis=0, keepdims=True)
        return carry

    jax.lax.fori_loop(0, _B, phase1, 0)

    scale = 1.0 / float(_B * N * K)

    def bn_relu(z, g_ref, beta_ref):
        C = z.shape[1]
        zbuf_ref[:, 0:C] = z

        def acc_stats(b, carry):
            s1, s2 = carry
            cb = cnt_ref[pl.ds(b, 1), :]
            zb = zbuf_ref[pl.ds(b * N, N), 0:C]
            s1 = s1 + jnp.dot(cb, zb, preferred_element_type=jnp.float32)
            s2 = s2 + jnp.dot(cb, zb * zb,
                              preferred_element_type=jnp.float32)
            return s1, s2

        s1, s2 = jax.lax.fori_loop(
            0, _B, acc_stats,
            (jnp.zeros((1, C), jnp.float32), jnp.zeros((1, C), jnp.float32)))
        mean = s1 * scale
        var = s2 * scale - mean * mean
        y = (z - mean) * jax.lax.rsqrt(var + _EPS)
        y = y * g_ref[:] + beta_ref[:]
        return jnp.maximum(y, 0.0)

    X = x_ref[:].reshape(_B * N, 8)
    z1 = jnp.dot(X, W1_ref[:].T, preferred_element_type=jnp.float32)
    y1 = bn_relu(z1, g1_ref, b1_ref)
    z2 = jnp.dot(y1, W2_ref[:].T, preferred_element_type=jnp.float32)
    y2 = bn_relu(z2, g2_ref, b2_ref)
    z3 = jnp.dot(y2, W3_ref[:].T, preferred_element_type=jnp.float32)
    y3 = bn_relu(z3, g3_ref, b3_ref)

    zbuf_ref[:] = y3
    fc1w = fc1w_ref[:]
    fc1b = fc1b_ref[:]
    fc2w = fc2w_ref[:]
    fc2b = fc2b_ref[:]

    def phase3(b, carry):
        xb = x_ref[pl.ds(b, 1)].reshape(N, 8)
        P = _pairwise(xb)
        y3b = zbuf_ref[pl.ds(b * N, N), :]

        def step(_, state):
            mask, acc = state
            cur = jnp.where(mask > 0.5, _NEG, P)
            vmax = jnp.max(cur, axis=1, keepdims=True)
            ii = jnp.min(jnp.where(cur >= vmax, iota, N), axis=1,
                         keepdims=True)
            onehot = (iota == ii).astype(jnp.float32)
            sel = jnp.dot(onehot, y3b, preferred_element_type=jnp.float32)
            return mask + onehot, jnp.maximum(acc, sel)

        _, acc = jax.lax.fori_loop(
            0, K, step,
            (jnp.zeros((N, N), jnp.float32), jnp.zeros((N, 256), jnp.float32)))
        h = jnp.dot(fc1w, acc, preferred_element_type=jnp.float32) + fc1b
        h = jnp.maximum(h, 0.0)
        o = jnp.dot(fc2w, h, preferred_element_type=jnp.float32) + fc2b
        out_ref[pl.ds(b, 1)] = o.reshape(1, 40, N)
        return carry

    jax.lax.fori_loop(0, _B, phase3, 0)


def kernel(x, W1, g1, b1, W2, g2, b2, W3, g3, b3, fc1_w, fc1_b, fc2_w, fc2_b):
    xp = jnp.pad(x, ((0, 0), (0, 0), (0, 5)))
    W1p = jnp.pad(W1, ((0, 0), (0, 5)))
    out = pl.pallas_call(
        _vndgcnn_kernel,
        out_shape=jax.ShapeDtypeStruct((_B, 40, _N), jnp.float32),
        scratch_shapes=[pltpu.VMEM((_B, _N), jnp.float32),
                        pltpu.VMEM((_B * _N, 256), jnp.float32)],
    )(xp,
      W1p, g1.reshape(1, -1), b1.reshape(1, -1),
      W2, g2.reshape(1, -1), b2.reshape(1, -1),
      W3, g3.reshape(1, -1), b3.reshape(1, -1),
      fc1_w, fc1_b.reshape(-1, 1), fc2_w, fc2_b.reshape(-1, 1))
    return jnp.transpose(out, (0, 2, 1))


# speedup vs baseline: 3.1375x; 1.4834x over previous
import jax
import jax.numpy as jnp
from jax.experimental import pallas as pl
from jax.experimental.pallas import tpu as pltpu

_B, _N, _K = 16, 256, 20
_EPS = 1e-5
_NEG = -3.0e38


def _pairwise(xb):
    inner = jnp.dot(xb, xb.T, preferred_element_type=jnp.float32)
    xx = jnp.sum(xb * xb, axis=1)
    return 2.0 * inner - xx[:, None] - xx[None, :]


def _vndgcnn_kernel(x_ref, W1_ref, g1_ref, b1_ref, W2_ref, g2_ref, b2_ref,
                    W3_ref, g3_ref, b3_ref, fc1w_ref, fc1b_ref, fc2w_ref,
                    fc2b_ref, out_ref, stamp_ref, ycat_ref):
    N, K = _N, _K
    iota_m = jax.lax.broadcasted_iota(jnp.int32, (N, N), 0)

    def phase1(b, carry):
        xb = x_ref[pl.ds(b, 1)].reshape(N, 8)
        P = _pairwise(xb)

        def step(i, st):
            cur, stamp = st
            vmax = jnp.max(cur, axis=0, keepdims=True)
            ii = jnp.min(jnp.where(cur >= vmax, iota_m, N), axis=0,
                         keepdims=True)
            oh = iota_m == ii
            cur = jnp.where(oh, _NEG, cur)
            stamp = stamp + oh.astype(jnp.float32) * (
                (i + 1).astype(jnp.float32))
            return cur, stamp

        _, stamp = jax.lax.fori_loop(
            0, K, step, (P, jnp.zeros((N, N), jnp.float32)))
        stamp_ref[pl.ds(b * N, N), :] = stamp
        return carry

    jax.lax.fori_loop(0, _B, phase1, 0)

    maskf = (stamp_ref[:] > 0.0).astype(jnp.float32)
    ones_col = jnp.ones((N, 1), jnp.float32)
    cnt = jnp.dot(maskf, ones_col, preferred_element_type=jnp.float32)

    scale = 1.0 / float(_B * N * K)
    tdims = (((0,), (0,)), ((), ()))

    def bn_relu(z, g_ref, beta_ref):
        s1 = jax.lax.dot_general(cnt, z, tdims,
                                 preferred_element_type=jnp.float32)
        s2 = jax.lax.dot_general(cnt, z * z, tdims,
                                 preferred_element_type=jnp.float32)
        mean = s1 * scale
        var = s2 * scale - mean * mean
        y = (z - mean) * jax.lax.rsqrt(var + _EPS)
        y = y * g_ref[:] + beta_ref[:]
        return jnp.maximum(y, 0.0)

    X = x_ref[:].reshape(_B * N, 8)
    z1 = jnp.dot(X, W1_ref[:].T, preferred_element_type=jnp.float32)
    y1 = bn_relu(z1, g1_ref, b1_ref)
    z2 = jnp.dot(y1, W2_ref[:].T, preferred_element_type=jnp.float32)
    y2 = bn_relu(z2, g2_ref, b2_ref)
    z3 = jnp.dot(y2, W3_ref[:].T, preferred_element_type=jnp.float32)
    y3 = bn_relu(z3, g3_ref, b3_ref)

    y_hi = y3.astype(jnp.bfloat16)
    y_lo = (y3 - y_hi.astype(jnp.float32)).astype(jnp.bfloat16)
    ycat_ref[:] = jnp.concatenate([y_hi, y_lo], axis=1)

    fc1w = fc1w_ref[:]
    fc1b = fc1b_ref[:]
    fc2w = fc2w_ref[:]
    fc2b = fc2b_ref[:]

    def phase3(b, carry):
        stamp_b = stamp_ref[pl.ds(b * N, N), :]
        ycat_b = ycat_ref[pl.ds(b * N, N), :]

        def step(i, acc):
            oh = (stamp_b == (i + 1).astype(jnp.float32))
            sel2 = jax.lax.dot_general(
                oh.astype(jnp.bfloat16), ycat_b, tdims,
                preferred_element_type=jnp.float32)
            sel = sel2[:, 0:256] + sel2[:, 256:512]
            return jnp.maximum(acc, sel)

        acc = jax.lax.fori_loop(0, K, step, jnp.zeros((N, 256), jnp.float32))
        h = jnp.dot(fc1w, acc, preferred_element_type=jnp.float32) + fc1b
        h = jnp.maximum(h, 0.0)
        o = jnp.dot(fc2w, h, preferred_element_type=jnp.float32) + fc2b
        out_ref[pl.ds(b, 1)] = o.reshape(1, 40, N)
        return carry

    jax.lax.fori_loop(0, _B, phase3, 0)


def kernel(x, W1, g1, b1, W2, g2, b2, W3, g3, b3, fc1_w, fc1_b, fc2_w, fc2_b):
    xp = jnp.pad(x, ((0, 0), (0, 0), (0, 5)))
    W1p = jnp.pad(W1, ((0, 0), (0, 5)))
    out = pl.pallas_call(
        _vndgcnn_kernel,
        out_shape=jax.ShapeDtypeStruct((_B, 40, _N), jnp.float32),
        scratch_shapes=[pltpu.VMEM((_B * _N, _N), jnp.float32),
                        pltpu.VMEM((_B * _N, 512), jnp.bfloat16)],
    )(xp,
      W1p, g1.reshape(1, -1), b1.reshape(1, -1),
      W2, g2.reshape(1, -1), b2.reshape(1, -1),
      W3, g3.reshape(1, -1), b3.reshape(1, -1),
      fc1_w, fc1_b.reshape(-1, 1), fc2_w, fc2_b.reshape(-1, 1))
    return jnp.transpose(out, (0, 2, 1))


# speedup vs baseline: 3.5900x; 1.1442x over previous
import jax
import jax.numpy as jnp
from jax.experimental import pallas as pl
from jax.experimental.pallas import tpu as pltpu

_B, _N, _K = 16, 256, 20
_EPS = 1e-5
_NEG = -3.0e38


def _pairwise(xb):
    inner = jnp.dot(xb, xb.T, preferred_element_type=jnp.float32)
    xx = jnp.sum(xb * xb, axis=1)
    return 2.0 * inner - xx[:, None] - xx[None, :]


def _vndgcnn_kernel(x_ref, W1_ref, g1_ref, b1_ref, W2_ref, g2_ref, b2_ref,
                    W3_ref, g3_ref, b3_ref, fc1w_ref, fc1b_ref, fc2w_ref,
                    fc2b_ref, out_ref, idx_ref, cnt_ref, ycat_ref):
    N, K = _N, _K
    iota_m = jax.lax.broadcasted_iota(jnp.int32, (N, N), 0)
    ones_col = jnp.ones((N, 1), jnp.float32)

    def phase1(b, carry):
        xb = x_ref[pl.ds(b, 1)].reshape(N, 8)
        P = _pairwise(xb)

        def step(i, cur):
            vmax = jnp.max(cur, axis=0, keepdims=True)
            ii = jnp.min(jnp.where(cur >= vmax, iota_m, N), axis=0,
                         keepdims=True)
            idx_ref[pl.ds(b * K + i, 1), :] = ii
            return jnp.where(iota_m == ii, _NEG, cur)

        cur = jax.lax.fori_loop(0, K, step, P)
        maskf = (cur <= -1.0e37).astype(jnp.float32)
        cnt_ref[pl.ds(b * N, N), :] = jnp.dot(
            maskf, jnp.ones((N, 8), jnp.float32),
            preferred_element_type=jnp.float32)
        return carry

    jax.lax.fori_loop(0, _B, phase1, 0)

    cnt = cnt_ref[:, 0:1]

    scale = 1.0 / float(_B * N * K)
    tdims = (((0,), (0,)), ((), ()))

    def bn_relu(z, g_ref, beta_ref):
        s1 = jax.lax.dot_general(cnt, z, tdims,
                                 preferred_element_type=jnp.float32)
        s2 = jax.lax.dot_general(cnt, z * z, tdims,
                                 preferred_element_type=jnp.float32)
        mean = s1 * scale
        var = s2 * scale - mean * mean
        y = (z - mean) * jax.lax.rsqrt(var + _EPS)
        y = y * g_ref[:] + beta_ref[:]
        return jnp.maximum(y, 0.0)

    X = x_ref[:].reshape(_B * N, 8)
    z1 = jnp.dot(X, W1_ref[:].T, preferred_element_type=jnp.float32)
    y1 = bn_relu(z1, g1_ref, b1_ref)
    z2 = jnp.dot(y1, W2_ref[:].T, preferred_element_type=jnp.float32)
    y2 = bn_relu(z2, g2_ref, b2_ref)
    z3 = jnp.dot(y2, W3_ref[:].T, preferred_element_type=jnp.float32)
    y3 = bn_relu(z3, g3_ref, b3_ref)

    y_hi = y3.astype(jnp.bfloat16)
    y_lo = (y3 - y_hi.astype(jnp.float32)).astype(jnp.bfloat16)
    ycat_ref[:] = jnp.concatenate([y_hi, y_lo], axis=1)

    fc1w = fc1w_ref[:]
    fc1b = fc1b_ref[:]
    fc2w = fc2w_ref[:]
    fc2b = fc2b_ref[:]

    def phase3(b, carry):
        ycat_b = ycat_ref[pl.ds(b * N, N), :]

        def step(i, acc):
            ii_row = idx_ref[pl.ds(b * K + i, 1), :]
            oh = iota_m == ii_row
            sel2 = jax.lax.dot_general(
                oh.astype(jnp.bfloat16), ycat_b, tdims,
                preferred_element_type=jnp.float32)
            sel = sel2[:, 0:256] + sel2[:, 256:512]
            return jnp.maximum(acc, sel)

        acc = jax.lax.fori_loop(0, K, step, jnp.zeros((N, 256), jnp.float32))
        h = jnp.dot(fc1w, acc, preferred_element_type=jnp.float32) + fc1b
        h = jnp.maximum(h, 0.0)
        o = jnp.dot(fc2w, h, preferred_element_type=jnp.float32) + fc2b
        out_ref[pl.ds(b, 1)] = o.reshape(1, 40, N)
        return carry

    jax.lax.fori_loop(0, _B, phase3, 0)


def kernel(x, W1, g1, b1, W2, g2, b2, W3, g3, b3, fc1_w, fc1_b, fc2_w, fc2_b):
    xp = jnp.pad(x, ((0, 0), (0, 0), (0, 5)))
    W1p = jnp.pad(W1, ((0, 0), (0, 5)))
    out = pl.pallas_call(
        _vndgcnn_kernel,
        out_shape=jax.ShapeDtypeStruct((_B, 40, _N), jnp.float32),
        scratch_shapes=[pltpu.VMEM((_B * _K, _N), jnp.int32),
                        pltpu.VMEM((_B * _N, 8), jnp.float32),
                        pltpu.VMEM((_B * _N, 512), jnp.bfloat16)],
    )(xp,
      W1p, g1.reshape(1, -1), b1.reshape(1, -1),
      W2, g2.reshape(1, -1), b2.reshape(1, -1),
      W3, g3.reshape(1, -1), b3.reshape(1, -1),
      fc1_w, fc1_b.reshape(-1, 1), fc2_w, fc2_b.reshape(-1, 1))
    return jnp.transpose(out, (0, 2, 1))
